# baseline (speedup 1.0000x reference)
import jax
import jax.numpy as jnp
from jax import lax
from jax.experimental import pallas as pl
from jax.experimental.pallas import tpu as pltpu
from jax.experimental.pallas import tpu_sc as plsc

SN = 128
S = 50
SP = 64
RR = 4
CROP = 96
NC, NS = 2, 16
NW = NC * NS
SPT = SN // NW
L = 16
NCH = SP // L


def _sload(ref, i):
    return ref[pl.ds(i, L)][0]


def _sc_body(ev_hbm, tt_hbm, len_hbm, st_hbm, out_hbm,
             ev, ttin, lenv, stv, loc, mmv, vals, snap,
             slabs, sems):
    wid = lax.axis_index("s") * NC + lax.axis_index("c")
    base = wid * SPT

    pltpu.sync_copy(ev_hbm.at[pl.ds(base * S * 4, SPT * S * 4)],
                    ev.at[pl.ds(0, SPT * S * 4)])
    pltpu.sync_copy(tt_hbm.at[pl.ds(base * S, SPT * S)],
                    ttin.at[pl.ds(0, SPT * S)])
    pltpu.sync_copy(len_hbm, lenv.at[pl.ds(0, SN)])
    pltpu.sync_copy(st_hbm, stv)

    zero16f = jnp.zeros((L,), jnp.float32)

    def zrow(row, carry):
        for b in range(2):
            for p in range(2):
                for k in range(CROP // L):
                    slabs[b, p, row, pl.ds(k * L, L)] = zero16f
        return carry
    lax.fori_loop(0, CROP, zrow, 0)

    start = _sload(stv, 0)
    lane = jnp.arange(L, dtype=jnp.int32)

    def sample_body(si, carry):
        eb = si * S * 4

        locc, pvc, xcc, ycc, mvc = [], [], [], [], []
        for k in range(NCH):
            ix = eb + 4 * (k * L + lane)
            in_s = (k * L + lane) < S
            ixc = jnp.where(in_s, ix, eb)
            x = plsc.load_gather(ev, [ixc]).astype(jnp.int32)
            y = plsc.load_gather(ev, [ixc + 1]).astype(jnp.int32)
            p = plsc.load_gather(ev, [ixc + 2]).astype(jnp.int32)
            lc = jnp.where(in_s, (p * 128 + x) * 128 + y, -1)
            loc[pl.ds(k * L, L)] = lc
            locc.append(lc)
            xs = x - start
            ys = y - start
            ok = (in_s & (xs >= 0) & (xs < CROP) & (ys >= 0) & (ys < CROP))
            pvc.append(p)
            xcc.append(jnp.clip(xs, 0, CROP - 1))
            ycc.append(jnp.clip(ys, 0, CROP - 1))
            mvc.append(ok)
            t = plsc.load_gather(ev, [ixc + 3])
            tp = plsc.load_gather(ev, [jnp.maximum(ixc - 4, eb) + 3])
            mmv[pl.ds(k * L, L)] = jnp.exp((tp - t) / jnp.float32(50.0))

        loc0 = _sload(loc, 0)
        tt0 = _sload(ttin, si * S)
        init0 = jnp.where(tt0 == 0, jnp.float32(0.3), jnp.float32(0.0))
        for k in range(NCH):
            m0 = locc[k] == loc0
            vals[pl.ds(k * L, L)] = jnp.where(m0, init0, 0.0)
            snap[pl.ds(k * L, L)] = jnp.where(m0, jnp.float32(0.3), 0.0)
            for r in range(1, RR):
                snap[pl.ds(r * SP + k * L, L)] = zero16f

        lenval = _sload(lenv, base + si)

        def step(n, carry):
            mmn = _sload(mmv, n)
            locn = _sload(loc, n)
            ttn = _sload(ttin, si * S + n)
            g = jnp.where(lenval >= n, jnp.float32(0.3), jnp.float32(0.0))
            c = _sload(vals, n) * mmn
            delta = g * (jnp.float32(1.0) - c)
            sb = ttn * SP
            for k in range(NCH):
                sl = pl.ds(k * L, L)
                v = vals[sl] * mmn + jnp.where(loc[sl] == locn, delta, 0.0)
                vals[sl] = v
                snap[pl.ds(sb + k * L, L)] = v
            return carry
        lax.fori_loop(1, S, step, 0)

        gi = base + si

        def fill(r, b):
            for k in range(NCH):
                vvec = snap[pl.ds(r * SP + k * L, L)]
                plsc.store_scatter(slabs.at[b], [pvc[k], xcc[k], ycc[k]],
                                   vvec, mask=mvc[k])
            return pltpu.async_copy(slabs.at[b], out_hbm.at[gi, r],
                                    sems.at[b])

        def reset(b):
            for k in range(NCH):
                plsc.store_scatter(slabs.at[b], [pvc[k], xcc[k], ycc[k]],
                                   zero16f, mask=mvc[k])

        h0 = fill(0, 0)
        h1 = fill(1, 1)
        h0.wait()
        reset(0)
        h2 = fill(2, 0)
        h1.wait()
        reset(1)
        h3 = fill(3, 1)
        h2.wait()
        reset(0)
        h3.wait()
        reset(1)
        return carry

    lax.fori_loop(0, SPT, sample_body, 0)


@jax.jit
def _run(event_flat, tt_flat, length, start_arr):
    mesh = plsc.VectorSubcoreMesh(core_axis_name="c", subcore_axis_name="s",
                                  num_cores=NC, num_subcores=NS)
    f = pl.kernel(
        _sc_body,
        out_type=jax.ShapeDtypeStruct((SN, RR, 2, CROP, CROP), jnp.float32),
        mesh=mesh,
        compiler_params=pltpu.CompilerParams(needs_layout_passes=False),
        scratch_types=[
            pltpu.VMEM((SPT * S * 4 + L,), jnp.float32),
            pltpu.VMEM((SPT * S + L,), jnp.int32),
            pltpu.VMEM((SN + L,), jnp.int32),
            pltpu.VMEM((L,), jnp.int32),
            pltpu.VMEM((SP + L,), jnp.int32),
            pltpu.VMEM((SP + L,), jnp.float32),
            pltpu.VMEM((SP + L,), jnp.float32),
            pltpu.VMEM((RR * SP,), jnp.float32),
            pltpu.VMEM((2, 2, CROP, CROP), jnp.float32),
            pltpu.SemaphoreType.DMA((2,)),
        ],
    )
    return f(event_flat, tt_flat, length, start_arr)


def kernel(event, time_trace, length, test):
    ev_flat = event.astype(jnp.float32).reshape(-1)
    tt_flat = time_trace.astype(jnp.int32).reshape(-1)
    ln = length.astype(jnp.int32)
    start = jnp.where(jnp.asarray(test, dtype=jnp.int32) == 1, 26, 16)
    start_arr = jnp.broadcast_to(start.astype(jnp.int32), (L,))
    return _run(ev_flat, tt_flat, ln, start_arr)


# speedup vs baseline: 1.0460x; 1.0460x over previous
import jax
import jax.numpy as jnp
from jax import lax
from jax.experimental import pallas as pl
from jax.experimental.pallas import tpu as pltpu
from jax.experimental.pallas import tpu_sc as plsc

SN = 128
S = 50
SP = 64
RR = 4
CROP = 96
NC, NS = 2, 16
NW = NC * NS
SPT = SN // NW
L = 16
NCH = SP // L


def _sload(ref, i):
    return ref[pl.ds(i, L)][0]


def _sc_body(ev_hbm, tt_hbm, len_hbm, st_hbm, out_hbm,
             ev, ttin, lenv, stv, loc, mmv, vals, snap,
             slabs, sems):
    wid = lax.axis_index("s") * NC + lax.axis_index("c")
    base = wid * SPT

    pltpu.sync_copy(ev_hbm.at[pl.ds(base * S * 4, SPT * S * 4)],
                    ev.at[pl.ds(0, SPT * S * 4)])
    pltpu.sync_copy(tt_hbm.at[pl.ds(base * S, SPT * S)],
                    ttin.at[pl.ds(0, SPT * S)])
    pltpu.sync_copy(len_hbm, lenv.at[pl.ds(0, SN)])
    pltpu.sync_copy(st_hbm, stv)

    zero16f = jnp.zeros((L,), jnp.float32)

    def zrow(row, carry):
        for b in range(2):
            for p in range(2):
                for k in range(CROP // L):
                    slabs[b, p, row, pl.ds(k * L, L)] = zero16f
        return carry
    lax.fori_loop(0, CROP, zrow, 0)

    start = _sload(stv, 0)
    lane = jnp.arange(L, dtype=jnp.int32)
    pending = [None, None]

    for si in range(SPT):
        eb = si * S * 4

        locc, pvc, xcc, ycc, mvc = [], [], [], [], []
        for k in range(NCH):
            ix = eb + 4 * (k * L + lane)
            in_s = (k * L + lane) < S
            ixc = jnp.where(in_s, ix, eb)
            x = plsc.load_gather(ev, [ixc]).astype(jnp.int32)
            y = plsc.load_gather(ev, [ixc + 1]).astype(jnp.int32)
            p = plsc.load_gather(ev, [ixc + 2]).astype(jnp.int32)
            lc = jnp.where(in_s, (p * 128 + x) * 128 + y, -1)
            loc[pl.ds(k * L, L)] = lc
            locc.append(lc)
            xs = x - start
            ys = y - start
            ok = (in_s & (xs >= 0) & (xs < CROP) & (ys >= 0) & (ys < CROP))
            pvc.append(p)
            xcc.append(jnp.clip(xs, 0, CROP - 1))
            ycc.append(jnp.clip(ys, 0, CROP - 1))
            mvc.append(ok)
            t = plsc.load_gather(ev, [ixc + 3])
            tp = plsc.load_gather(ev, [jnp.maximum(ixc - 4, eb) + 3])
            mmv[pl.ds(k * L, L)] = jnp.exp((tp - t) / jnp.float32(50.0))

        loc0 = _sload(loc, 0)
        tt0 = _sload(ttin, si * S)
        init0 = jnp.where(tt0 == 0, jnp.float32(0.3), jnp.float32(0.0))
        for k in range(NCH):
            m0 = locc[k] == loc0
            vals[pl.ds(k * L, L)] = jnp.where(m0, init0, 0.0)
            snap[pl.ds(k * L, L)] = jnp.where(m0, jnp.float32(0.3), 0.0)
            for r in range(1, RR):
                snap[pl.ds(r * SP + k * L, L)] = zero16f

        lenval = _sload(lenv, base + si)

        def step(n, carry):
            mmn = _sload(mmv, n)
            locn = _sload(loc, n)
            ttn = _sload(ttin, si * S + n)
            g = jnp.where(lenval >= n, jnp.float32(0.3), jnp.float32(0.0))
            c = _sload(vals, n) * mmn
            delta = g * (jnp.float32(1.0) - c)
            sb = ttn * SP
            for k in range(NCH):
                sl = pl.ds(k * L, L)
                v = vals[sl] * mmn + jnp.where(loc[sl] == locn, delta, 0.0)
                vals[sl] = v
                snap[pl.ds(sb + k * L, L)] = v
            return carry
        lax.fori_loop(1, S, step, 0)

        gi = base + si
        for r in range(RR):
            b = r % 2
            if pending[b] is not None:
                h, rst = pending[b]
                h.wait()
                for (pv, xv, yv, mv) in rst:
                    plsc.store_scatter(slabs.at[b], [pv, xv, yv],
                                       zero16f, mask=mv)
            for k in range(NCH):
                vvec = snap[pl.ds(r * SP + k * L, L)]
                plsc.store_scatter(slabs.at[b], [pvc[k], xcc[k], ycc[k]],
                                   vvec, mask=mvc[k])
            h = pltpu.async_copy(slabs.at[b], out_hbm.at[gi, r], sems.at[b])
            pending[b] = (h, list(zip(pvc, xcc, ycc, mvc)))

    for b in range(2):
        h, _ = pending[b]
        h.wait()


@jax.jit
def _run(event_flat, tt_flat, length, start_arr):
    mesh = plsc.VectorSubcoreMesh(core_axis_name="c", subcore_axis_name="s",
                                  num_cores=NC, num_subcores=NS)
    f = pl.kernel(
        _sc_body,
        out_type=jax.ShapeDtypeStruct((SN, RR, 2, CROP, CROP), jnp.float32),
        mesh=mesh,
        compiler_params=pltpu.CompilerParams(needs_layout_passes=False,
                                             use_tc_tiling_on_sc=True),
        scratch_types=[
            pltpu.VMEM((SPT * S * 4 + L,), jnp.float32),
            pltpu.VMEM((SPT * S + L,), jnp.int32),
            pltpu.VMEM((SN + L,), jnp.int32),
            pltpu.VMEM((L,), jnp.int32),
            pltpu.VMEM((SP + L,), jnp.int32),
            pltpu.VMEM((SP + L,), jnp.float32),
            pltpu.VMEM((SP + L,), jnp.float32),
            pltpu.VMEM((RR * SP,), jnp.float32),
            pltpu.VMEM((2, 2, CROP, CROP), jnp.float32),
            pltpu.SemaphoreType.DMA((2,)),
        ],
    )
    return f(event_flat, tt_flat, length, start_arr)


def kernel(event, time_trace, length, test):
    ev_flat = event.astype(jnp.float32).reshape(-1)
    tt_flat = time_trace.astype(jnp.int32).reshape(-1)
    ln = length.astype(jnp.int32)
    start = jnp.where(jnp.asarray(test, dtype=jnp.int32) == 1, 26, 16)
    start_arr = jnp.broadcast_to(start.astype(jnp.int32), (L,))
    return _run(ev_flat, tt_flat, ln, start_arr)
